# baseline (speedup 1.0000x reference)
import jax
import jax.numpy as jnp
from jax import lax
from jax.experimental import pallas as pl
from jax.experimental.pallas import tpu as pltpu
from jax.experimental.pallas import tpu_sc as plsc

N_NODES = 100000
N_EDGES = 6400000
NC = 2
NS = 16
L = 16

CHUNK = 3200
N_CH = N_EDGES // CHUNK
T_SCORE = 12
T_PN = NC * NS - T_SCORE


def _pipeline(n_r, base_of, start_in, wait_in, compute, start_out, wait_out):
    start_in(0, 0)
    n_half = (n_r + 1) // 2

    def it(h, carry):
        i0 = 2 * h
        i1 = i0 + 1

        @pl.when(i1 < n_r)
        def _():
            start_in(1, i1)

        wait_in(0)

        @pl.when(i0 >= 2)
        def _():
            wait_out(0)

        compute(0, base_of(i0))
        start_out(0, i0)

        @pl.when(i0 + 2 < n_r)
        def _():
            start_in(0, i0 + 2)

        @pl.when(i1 < n_r)
        def _():
            wait_in(1)

            @pl.when(i1 >= 2)
            def _():
                wait_out(1)

            compute(1, base_of(i1))
            start_out(1, i1)

        return carry

    lax.fori_loop(0, n_half, it, 0)
    wait_out(0)

    @pl.when(n_r >= 2)
    def _():
        wait_out(1)


def _body(scores_hbm, pn_hbm, edges_hbm, score_out, mask_out, table_v):
    c = lax.axis_index("c")
    s = lax.axis_index("s")
    wid = s * NC + c
    is_score = wid < T_SCORE

    @pl.when(is_score)
    def _():
        r = wid
        n_r = (N_CH - r - 1) // T_SCORE + 1

        def base_of(i):
            return (r + i * T_SCORE) * CHUNK

        def scoped(idx, out, tsem, si, so):
            tcopy = pltpu.async_copy(scores_hbm, table_v, tsem)

            def start_in(sl, i):
                pltpu.async_copy(edges_hbm.at[pl.ds(base_of(i), CHUNK)],
                                 idx[sl], si[sl])

            def wait_in(sl):
                pltpu.make_async_copy(edges_hbm.at[pl.ds(0, CHUNK)],
                                      idx[sl], si[sl]).wait()

            def compute(sl, _base):
                idx_v, out_v = idx[sl], out[sl]

                @plsc.parallel_loop(0, CHUNK, step=L, unroll=16)
                def _grp(off):
                    sl2 = pl.ds(off, L)
                    vals = plsc.load_gather(table_v, [idx_v[sl2]])
                    out_v[sl2] = plsc.bitcast(vals, jnp.float32)

            def start_out(sl, i):
                pltpu.async_copy(out[sl],
                                 score_out.at[pl.ds(base_of(i), CHUNK)], so[sl])

            def wait_out(sl):
                pltpu.make_async_copy(out[sl],
                                      score_out.at[pl.ds(0, CHUNK)],
                                      so[sl]).wait()

            tcopy.wait()
            _pipeline(n_r, base_of, start_in, wait_in, compute, start_out,
                      wait_out)

        pl.run_scoped(
            lambda i0, i1, o0, o1, tsem, si0, si1, so0, so1: scoped(
                (i0, i1), (o0, o1), tsem, (si0, si1), (so0, so1)),
            pltpu.VMEM((CHUNK,), jnp.int32), pltpu.VMEM((CHUNK,), jnp.int32),
            pltpu.VMEM((CHUNK,), jnp.float32), pltpu.VMEM((CHUNK,), jnp.float32),
            pltpu.SemaphoreType.DMA, pltpu.SemaphoreType.DMA,
            pltpu.SemaphoreType.DMA, pltpu.SemaphoreType.DMA,
            pltpu.SemaphoreType.DMA)

    @pl.when(jnp.logical_not(is_score))
    def _():
        r = wid - T_SCORE
        n_r = (N_CH - r - 1) // T_PN + 1

        def base_of(i):
            return (r + i * T_PN) * CHUNK

        def scoped(idx_a, idx_b, out, tsem, si, so):
            tcopy = pltpu.async_copy(pn_hbm, table_v, tsem)

            def start_in(sl, i):
                pltpu.async_copy(edges_hbm.at[pl.ds(base_of(i), CHUNK)],
                                 idx_a[sl], si[sl])
                pltpu.async_copy(
                    edges_hbm.at[pl.ds(N_EDGES + base_of(i), CHUNK)],
                    idx_b[sl], si[sl])

            def wait_in(sl):
                pltpu.make_async_copy(edges_hbm.at[pl.ds(0, CHUNK)],
                                      idx_a[sl], si[sl]).wait()
                pltpu.make_async_copy(edges_hbm.at[pl.ds(0, CHUNK)],
                                      idx_b[sl], si[sl]).wait()

            def compute(sl, _base):
                ia_v, ib_v, out_v = idx_a[sl], idx_b[sl], out[sl]

                @plsc.parallel_loop(0, CHUNK, step=L, unroll=16)
                def _grp(off):
                    sl2 = pl.ds(off, L)
                    va = plsc.load_gather(table_v, [ia_v[sl2]])
                    vb = plsc.load_gather(table_v, [ib_v[sl2]])
                    out_v[sl2] = (va == vb).astype(jnp.int32)

            def start_out(sl, i):
                pltpu.async_copy(out[sl],
                                 mask_out.at[pl.ds(base_of(i), CHUNK)], so[sl])

            def wait_out(sl):
                pltpu.make_async_copy(out[sl],
                                      mask_out.at[pl.ds(0, CHUNK)],
                                      so[sl]).wait()

            tcopy.wait()
            _pipeline(n_r, base_of, start_in, wait_in, compute, start_out,
                      wait_out)

        pl.run_scoped(
            lambda a0, a1, b0, b1, o0, o1, tsem, si0, si1, so0, so1: scoped(
                (a0, a1), (b0, b1), (o0, o1), tsem, (si0, si1), (so0, so1)),
            pltpu.VMEM((CHUNK,), jnp.int32), pltpu.VMEM((CHUNK,), jnp.int32),
            pltpu.VMEM((CHUNK,), jnp.int32), pltpu.VMEM((CHUNK,), jnp.int32),
            pltpu.VMEM((CHUNK,), jnp.int32), pltpu.VMEM((CHUNK,), jnp.int32),
            pltpu.SemaphoreType.DMA, pltpu.SemaphoreType.DMA,
            pltpu.SemaphoreType.DMA, pltpu.SemaphoreType.DMA,
            pltpu.SemaphoreType.DMA)


_sc_call = pl.kernel(
    _body,
    out_type=[
        jax.ShapeDtypeStruct((N_EDGES,), jnp.float32),
        jax.ShapeDtypeStruct((N_EDGES,), jnp.int32),
    ],
    mesh=plsc.VectorSubcoreMesh(core_axis_name="c", subcore_axis_name="s",
                                num_cores=NC, num_subcores=NS),
    scratch_types=[
        pltpu.VMEM((N_NODES,), jnp.int32),
    ],
    compiler_params=pltpu.CompilerParams(needs_layout_passes=False),
)


def kernel(scores, particle_number, edge_index):
    scores_i32 = lax.bitcast_convert_type(scores.reshape(-1), jnp.int32)
    pn_i32 = particle_number.astype(jnp.int32)
    score_neigh, mask_i32 = _sc_call(scores_i32, pn_i32, edge_index.reshape(-1))
    return score_neigh, mask_i32.astype(jnp.bool_)


# speedup vs baseline: 1.0299x; 1.0299x over previous
import jax
import jax.numpy as jnp
from jax import lax
from jax.experimental import pallas as pl
from jax.experimental.pallas import tpu as pltpu
from jax.experimental.pallas import tpu_sc as plsc

N_NODES = 100000
N_EDGES = 6400000
NC = 2
NS = 16
L = 16

CHUNK = 3200
N_CH = N_EDGES // CHUNK
T_SCORE = 12
T_PN = NC * NS - T_SCORE


def _pipeline(n_r, base_of, start_in, wait_in, compute, start_out, wait_out):
    start_in(0, 0)
    n_half = (n_r + 1) // 2

    def it(h, carry):
        i0 = 2 * h
        i1 = i0 + 1

        @pl.when(i1 < n_r)
        def _():
            start_in(1, i1)

        wait_in(0)

        @pl.when(i0 >= 2)
        def _():
            wait_out(0)

        compute(0, base_of(i0))
        start_out(0, i0)

        @pl.when(i0 + 2 < n_r)
        def _():
            start_in(0, i0 + 2)

        @pl.when(i1 < n_r)
        def _():
            wait_in(1)

            @pl.when(i1 >= 2)
            def _():
                wait_out(1)

            compute(1, base_of(i1))
            start_out(1, i1)

        return carry

    lax.fori_loop(0, n_half, it, 0)
    wait_out(0)

    @pl.when(n_r >= 2)
    def _():
        wait_out(1)


def _body(scores_hbm, pn_hbm, edges_hbm, score_out, mask_out, table_v):
    c = lax.axis_index("c")
    s = lax.axis_index("s")
    wid = s * NC + c
    is_score = wid < T_SCORE

    @pl.when(is_score)
    def _():
        r = wid
        n_r = (N_CH - r - 1) // T_SCORE + 1

        def base_of(i):
            return (r + i * T_SCORE) * CHUNK

        def scoped(idx, out, tsem, si, so):
            tcopy = pltpu.async_copy(scores_hbm, table_v, tsem)

            def start_in(sl, i):
                pltpu.async_copy(edges_hbm.at[pl.ds(base_of(i), CHUNK)],
                                 idx[sl], si[sl])

            def wait_in(sl):
                pltpu.make_async_copy(edges_hbm.at[pl.ds(0, CHUNK)],
                                      idx[sl], si[sl]).wait()

            def compute(sl, _base):
                idx_v, out_v = idx[sl], out[sl]

                @plsc.parallel_loop(0, CHUNK, step=L, unroll=8)
                def _grp(off):
                    sl2 = pl.ds(off, L)
                    vals = plsc.load_gather(table_v, [idx_v[sl2]])
                    out_v[sl2] = plsc.bitcast(vals, jnp.float32)

            def start_out(sl, i):
                pltpu.async_copy(out[sl],
                                 score_out.at[pl.ds(base_of(i), CHUNK)], so[sl])

            def wait_out(sl):
                pltpu.make_async_copy(out[sl],
                                      score_out.at[pl.ds(0, CHUNK)],
                                      so[sl]).wait()

            tcopy.wait()
            _pipeline(n_r, base_of, start_in, wait_in, compute, start_out,
                      wait_out)

        pl.run_scoped(
            lambda i0, i1, o0, o1, tsem, si0, si1, so0, so1: scoped(
                (i0, i1), (o0, o1), tsem, (si0, si1), (so0, so1)),
            pltpu.VMEM((CHUNK,), jnp.int32), pltpu.VMEM((CHUNK,), jnp.int32),
            pltpu.VMEM((CHUNK,), jnp.float32), pltpu.VMEM((CHUNK,), jnp.float32),
            pltpu.SemaphoreType.DMA, pltpu.SemaphoreType.DMA,
            pltpu.SemaphoreType.DMA, pltpu.SemaphoreType.DMA,
            pltpu.SemaphoreType.DMA)

    @pl.when(jnp.logical_not(is_score))
    def _():
        r = wid - T_SCORE
        n_r = (N_CH - r - 1) // T_PN + 1

        def base_of(i):
            return (r + i * T_PN) * CHUNK

        def scoped(idx_a, idx_b, out, tsem, si, so):
            tcopy = pltpu.async_copy(pn_hbm, table_v, tsem)

            def start_in(sl, i):
                pltpu.async_copy(edges_hbm.at[pl.ds(base_of(i), CHUNK)],
                                 idx_a[sl], si[sl])
                pltpu.async_copy(
                    edges_hbm.at[pl.ds(N_EDGES + base_of(i), CHUNK)],
                    idx_b[sl], si[sl])

            def wait_in(sl):
                pltpu.make_async_copy(edges_hbm.at[pl.ds(0, CHUNK)],
                                      idx_a[sl], si[sl]).wait()
                pltpu.make_async_copy(edges_hbm.at[pl.ds(0, CHUNK)],
                                      idx_b[sl], si[sl]).wait()

            def compute(sl, _base):
                ia_v, ib_v, out_v = idx_a[sl], idx_b[sl], out[sl]

                @plsc.parallel_loop(0, CHUNK, step=L, unroll=8)
                def _grp(off):
                    sl2 = pl.ds(off, L)
                    va = plsc.load_gather(table_v, [ia_v[sl2]])
                    vb = plsc.load_gather(table_v, [ib_v[sl2]])
                    out_v[sl2] = (va == vb).astype(jnp.int32)

            def start_out(sl, i):
                pltpu.async_copy(out[sl],
                                 mask_out.at[pl.ds(base_of(i), CHUNK)], so[sl])

            def wait_out(sl):
                pltpu.make_async_copy(out[sl],
                                      mask_out.at[pl.ds(0, CHUNK)],
                                      so[sl]).wait()

            tcopy.wait()
            _pipeline(n_r, base_of, start_in, wait_in, compute, start_out,
                      wait_out)

        pl.run_scoped(
            lambda a0, a1, b0, b1, o0, o1, tsem, si0, si1, so0, so1: scoped(
                (a0, a1), (b0, b1), (o0, o1), tsem, (si0, si1), (so0, so1)),
            pltpu.VMEM((CHUNK,), jnp.int32), pltpu.VMEM((CHUNK,), jnp.int32),
            pltpu.VMEM((CHUNK,), jnp.int32), pltpu.VMEM((CHUNK,), jnp.int32),
            pltpu.VMEM((CHUNK,), jnp.int32), pltpu.VMEM((CHUNK,), jnp.int32),
            pltpu.SemaphoreType.DMA, pltpu.SemaphoreType.DMA,
            pltpu.SemaphoreType.DMA, pltpu.SemaphoreType.DMA,
            pltpu.SemaphoreType.DMA)


_sc_call = pl.kernel(
    _body,
    out_type=[
        jax.ShapeDtypeStruct((N_EDGES,), jnp.float32),
        jax.ShapeDtypeStruct((N_EDGES,), jnp.int32),
    ],
    mesh=plsc.VectorSubcoreMesh(core_axis_name="c", subcore_axis_name="s",
                                num_cores=NC, num_subcores=NS),
    scratch_types=[
        pltpu.VMEM((N_NODES,), jnp.int32),
    ],
    compiler_params=pltpu.CompilerParams(needs_layout_passes=False),
)


def kernel(scores, particle_number, edge_index):
    scores_i32 = lax.bitcast_convert_type(scores.reshape(-1), jnp.int32)
    pn_i32 = particle_number.astype(jnp.int32)
    score_neigh, mask_i32 = _sc_call(scores_i32, pn_i32, edge_index.reshape(-1))
    return score_neigh, mask_i32.astype(jnp.bool_)


# speedup vs baseline: 1.1364x; 1.1035x over previous
import jax
import jax.numpy as jnp
from jax import lax
from jax.experimental import pallas as pl
from jax.experimental.pallas import tpu as pltpu
from jax.experimental.pallas import tpu_sc as plsc

N_NODES = 100000
N_EDGES = 6400000
NC = 2
NS = 16
L = 16

CHUNK = 6400
N_CH = N_EDGES // CHUNK
CHUNK_P = 3200
N_CH_P = N_EDGES // CHUNK_P
T_SCORE = 12
T_PN = NC * NS - T_SCORE


def _pipeline(n_r, base_of, start_in, wait_in, compute, start_out, wait_out):
    start_in(0, 0)
    n_half = (n_r + 1) // 2

    def it(h, carry):
        i0 = 2 * h
        i1 = i0 + 1

        @pl.when(i1 < n_r)
        def _():
            start_in(1, i1)

        wait_in(0)

        @pl.when(i0 >= 2)
        def _():
            wait_out(0)

        compute(0, base_of(i0))
        start_out(0, i0)

        @pl.when(i0 + 2 < n_r)
        def _():
            start_in(0, i0 + 2)

        @pl.when(i1 < n_r)
        def _():
            wait_in(1)

            @pl.when(i1 >= 2)
            def _():
                wait_out(1)

            compute(1, base_of(i1))
            start_out(1, i1)

        return carry

    lax.fori_loop(0, n_half, it, 0)
    wait_out(0)

    @pl.when(n_r >= 2)
    def _():
        wait_out(1)


def _body(scores_hbm, pn_hbm, edges_hbm, score_out, mask_out, table_v):
    c = lax.axis_index("c")
    s = lax.axis_index("s")
    wid = s * NC + c
    is_score = wid < T_SCORE

    @pl.when(is_score)
    def _():
        r = wid
        n_r = (N_CH - r - 1) // T_SCORE + 1

        def base_of(i):
            return (r + i * T_SCORE) * CHUNK

        def scoped(idx, out, tsem, si, so):
            tcopy = pltpu.async_copy(scores_hbm, table_v, tsem)

            def start_in(sl, i):
                pltpu.async_copy(edges_hbm.at[pl.ds(base_of(i), CHUNK)],
                                 idx[sl], si[sl])

            def wait_in(sl):
                pltpu.make_async_copy(edges_hbm.at[pl.ds(0, CHUNK)],
                                      idx[sl], si[sl]).wait()

            def compute(sl, _base):
                idx_v, out_v = idx[sl], out[sl]

                @plsc.parallel_loop(0, CHUNK, step=L, unroll=8)
                def _grp(off):
                    sl2 = pl.ds(off, L)
                    vals = plsc.load_gather(table_v, [idx_v[sl2]])
                    out_v[sl2] = plsc.bitcast(vals, jnp.float32)

            def start_out(sl, i):
                pltpu.async_copy(out[sl],
                                 score_out.at[pl.ds(base_of(i), CHUNK)], so[sl])

            def wait_out(sl):
                pltpu.make_async_copy(out[sl],
                                      score_out.at[pl.ds(0, CHUNK)],
                                      so[sl]).wait()

            tcopy.wait()
            _pipeline(n_r, base_of, start_in, wait_in, compute, start_out,
                      wait_out)

        pl.run_scoped(
            lambda i0, i1, o0, o1, tsem, si0, si1, so0, so1: scoped(
                (i0, i1), (o0, o1), tsem, (si0, si1), (so0, so1)),
            pltpu.VMEM((CHUNK,), jnp.int32), pltpu.VMEM((CHUNK,), jnp.int32),
            pltpu.VMEM((CHUNK,), jnp.float32), pltpu.VMEM((CHUNK,), jnp.float32),
            pltpu.SemaphoreType.DMA, pltpu.SemaphoreType.DMA,
            pltpu.SemaphoreType.DMA, pltpu.SemaphoreType.DMA,
            pltpu.SemaphoreType.DMA)

    @pl.when(jnp.logical_not(is_score))
    def _():
        r = wid - T_SCORE
        n_r = (N_CH_P - r - 1) // T_PN + 1

        def base_of(i):
            return (r + i * T_PN) * CHUNK_P

        def scoped(idx_a, idx_b, out, tsem, si, so):
            tcopy = pltpu.async_copy(pn_hbm, table_v, tsem)

            def start_in(sl, i):
                pltpu.async_copy(edges_hbm.at[pl.ds(base_of(i), CHUNK_P)],
                                 idx_a[sl], si[sl])
                pltpu.async_copy(
                    edges_hbm.at[pl.ds(N_EDGES + base_of(i), CHUNK_P)],
                    idx_b[sl], si[sl])

            def wait_in(sl):
                pltpu.make_async_copy(edges_hbm.at[pl.ds(0, CHUNK_P)],
                                      idx_a[sl], si[sl]).wait()
                pltpu.make_async_copy(edges_hbm.at[pl.ds(0, CHUNK_P)],
                                      idx_b[sl], si[sl]).wait()

            def compute(sl, _base):
                ia_v, ib_v, out_v = idx_a[sl], idx_b[sl], out[sl]

                @plsc.parallel_loop(0, CHUNK_P, step=L, unroll=8)
                def _grp(off):
                    sl2 = pl.ds(off, L)
                    va = plsc.load_gather(table_v, [ia_v[sl2]])
                    vb = plsc.load_gather(table_v, [ib_v[sl2]])
                    out_v[sl2] = (va == vb).astype(jnp.int32)

            def start_out(sl, i):
                pltpu.async_copy(out[sl],
                                 mask_out.at[pl.ds(base_of(i), CHUNK_P)], so[sl])

            def wait_out(sl):
                pltpu.make_async_copy(out[sl],
                                      mask_out.at[pl.ds(0, CHUNK_P)],
                                      so[sl]).wait()

            tcopy.wait()
            _pipeline(n_r, base_of, start_in, wait_in, compute, start_out,
                      wait_out)

        pl.run_scoped(
            lambda a0, a1, b0, b1, o0, o1, tsem, si0, si1, so0, so1: scoped(
                (a0, a1), (b0, b1), (o0, o1), tsem, (si0, si1), (so0, so1)),
            pltpu.VMEM((CHUNK_P,), jnp.int32), pltpu.VMEM((CHUNK_P,), jnp.int32),
            pltpu.VMEM((CHUNK_P,), jnp.int32), pltpu.VMEM((CHUNK_P,), jnp.int32),
            pltpu.VMEM((CHUNK_P,), jnp.int32), pltpu.VMEM((CHUNK_P,), jnp.int32),
            pltpu.SemaphoreType.DMA, pltpu.SemaphoreType.DMA,
            pltpu.SemaphoreType.DMA, pltpu.SemaphoreType.DMA,
            pltpu.SemaphoreType.DMA)


_sc_call = pl.kernel(
    _body,
    out_type=[
        jax.ShapeDtypeStruct((N_EDGES,), jnp.float32),
        jax.ShapeDtypeStruct((N_EDGES,), jnp.int32),
    ],
    mesh=plsc.VectorSubcoreMesh(core_axis_name="c", subcore_axis_name="s",
                                num_cores=NC, num_subcores=NS),
    scratch_types=[
        pltpu.VMEM((N_NODES,), jnp.int32),
    ],
    compiler_params=pltpu.CompilerParams(needs_layout_passes=False),
)


def kernel(scores, particle_number, edge_index):
    scores_i32 = lax.bitcast_convert_type(scores.reshape(-1), jnp.int32)
    pn_i32 = particle_number.astype(jnp.int32)
    score_neigh, mask_i32 = _sc_call(scores_i32, pn_i32, edge_index.reshape(-1))
    return score_neigh, mask_i32.astype(jnp.bool_)


# speedup vs baseline: 1.2170x; 1.0709x over previous
import jax
import jax.numpy as jnp
from jax import lax
from jax.experimental import pallas as pl
from jax.experimental.pallas import tpu as pltpu
from jax.experimental.pallas import tpu_sc as plsc

N_NODES = 100000
N_EDGES = 6400000
NC = 2
NS = 16
L = 16

CHUNK = 6400
N_CH = N_EDGES // CHUNK
CHUNK_P = 5120
N_CH_P = N_EDGES // CHUNK_P
T_SCORE = 12
T_PN = NC * NS - T_SCORE


def _pipeline(n_r, base_of, start_in, wait_in, compute, start_out, wait_out):
    start_in(0, 0)
    n_half = (n_r + 1) // 2

    def it(h, carry):
        i0 = 2 * h
        i1 = i0 + 1

        @pl.when(i1 < n_r)
        def _():
            start_in(1, i1)

        wait_in(0)

        @pl.when(i0 >= 2)
        def _():
            wait_out(0)

        compute(0, base_of(i0))
        start_out(0, i0)

        @pl.when(i0 + 2 < n_r)
        def _():
            start_in(0, i0 + 2)

        @pl.when(i1 < n_r)
        def _():
            wait_in(1)

            @pl.when(i1 >= 2)
            def _():
                wait_out(1)

            compute(1, base_of(i1))
            start_out(1, i1)

        return carry

    lax.fori_loop(0, n_half, it, 0)
    wait_out(0)

    @pl.when(n_r >= 2)
    def _():
        wait_out(1)


def _body(scores_hbm, pn_hbm, edges_hbm, score_out, mask_out, table_v):
    c = lax.axis_index("c")
    s = lax.axis_index("s")
    wid = s * NC + c
    is_score = wid < T_SCORE

    @pl.when(is_score)
    def _():
        r = wid
        n_r = (N_CH - r - 1) // T_SCORE + 1

        def base_of(i):
            return (r + i * T_SCORE) * CHUNK

        def scoped(idx, out, tsem, si, so):
            tcopy = pltpu.async_copy(scores_hbm, table_v, tsem)

            def start_in(sl, i):
                pltpu.async_copy(edges_hbm.at[pl.ds(base_of(i), CHUNK)],
                                 idx[sl], si[sl])

            def wait_in(sl):
                pltpu.make_async_copy(edges_hbm.at[pl.ds(0, CHUNK)],
                                      idx[sl], si[sl]).wait()

            def compute(sl, _base):
                idx_v, out_v = idx[sl], out[sl]

                @plsc.parallel_loop(0, CHUNK, step=L, unroll=8)
                def _grp(off):
                    sl2 = pl.ds(off, L)
                    vals = plsc.load_gather(table_v, [idx_v[sl2]])
                    out_v[sl2] = plsc.bitcast(vals, jnp.float32)

            def start_out(sl, i):
                pltpu.async_copy(out[sl],
                                 score_out.at[pl.ds(base_of(i), CHUNK)], so[sl])

            def wait_out(sl):
                pltpu.make_async_copy(out[sl],
                                      score_out.at[pl.ds(0, CHUNK)],
                                      so[sl]).wait()

            tcopy.wait()
            _pipeline(n_r, base_of, start_in, wait_in, compute, start_out,
                      wait_out)

        pl.run_scoped(
            lambda i0, i1, o0, o1, tsem, si0, si1, so0, so1: scoped(
                (i0, i1), (o0, o1), tsem, (si0, si1), (so0, so1)),
            pltpu.VMEM((CHUNK,), jnp.int32), pltpu.VMEM((CHUNK,), jnp.int32),
            pltpu.VMEM((CHUNK,), jnp.float32), pltpu.VMEM((CHUNK,), jnp.float32),
            pltpu.SemaphoreType.DMA, pltpu.SemaphoreType.DMA,
            pltpu.SemaphoreType.DMA, pltpu.SemaphoreType.DMA,
            pltpu.SemaphoreType.DMA)

    @pl.when(jnp.logical_not(is_score))
    def _():
        r = wid - T_SCORE
        n_r = (N_CH_P - r - 1) // T_PN + 1

        def base_of(i):
            return (r + i * T_PN) * CHUNK_P

        def scoped(idx_a, idx_b, out, tsem, si, so):
            tcopy = pltpu.async_copy(pn_hbm, table_v, tsem)

            def start_in(sl, i):
                pltpu.async_copy(edges_hbm.at[pl.ds(base_of(i), CHUNK_P)],
                                 idx_a[sl], si[sl])
                pltpu.async_copy(
                    edges_hbm.at[pl.ds(N_EDGES + base_of(i), CHUNK_P)],
                    idx_b[sl], si[sl])

            def wait_in(sl):
                pltpu.make_async_copy(edges_hbm.at[pl.ds(0, CHUNK_P)],
                                      idx_a[sl], si[sl]).wait()
                pltpu.make_async_copy(edges_hbm.at[pl.ds(0, CHUNK_P)],
                                      idx_b[sl], si[sl]).wait()

            def compute(sl, _base):
                ia_v, ib_v, out_v = idx_a[sl], idx_b[sl], out[sl]

                @plsc.parallel_loop(0, CHUNK_P, step=L, unroll=8)
                def _grp(off):
                    sl2 = pl.ds(off, L)
                    va = plsc.load_gather(table_v, [ia_v[sl2]])
                    vb = plsc.load_gather(table_v, [ib_v[sl2]])
                    out_v[sl2] = (va == vb).astype(jnp.int32)

            def start_out(sl, i):
                pltpu.async_copy(out[sl],
                                 mask_out.at[pl.ds(base_of(i), CHUNK_P)], so[sl])

            def wait_out(sl):
                pltpu.make_async_copy(out[sl],
                                      mask_out.at[pl.ds(0, CHUNK_P)],
                                      so[sl]).wait()

            tcopy.wait()
            _pipeline(n_r, base_of, start_in, wait_in, compute, start_out,
                      wait_out)

        pl.run_scoped(
            lambda a0, a1, b0, b1, o0, o1, tsem, si0, si1, so0, so1: scoped(
                (a0, a1), (b0, b1), (o0, o1), tsem, (si0, si1), (so0, so1)),
            pltpu.VMEM((CHUNK_P,), jnp.int32), pltpu.VMEM((CHUNK_P,), jnp.int32),
            pltpu.VMEM((CHUNK_P,), jnp.int32), pltpu.VMEM((CHUNK_P,), jnp.int32),
            pltpu.VMEM((CHUNK_P,), jnp.int32), pltpu.VMEM((CHUNK_P,), jnp.int32),
            pltpu.SemaphoreType.DMA, pltpu.SemaphoreType.DMA,
            pltpu.SemaphoreType.DMA, pltpu.SemaphoreType.DMA,
            pltpu.SemaphoreType.DMA)


_sc_call = pl.kernel(
    _body,
    out_type=[
        jax.ShapeDtypeStruct((N_EDGES,), jnp.float32),
        jax.ShapeDtypeStruct((N_EDGES,), jnp.int32),
    ],
    mesh=plsc.VectorSubcoreMesh(core_axis_name="c", subcore_axis_name="s",
                                num_cores=NC, num_subcores=NS),
    scratch_types=[
        pltpu.VMEM((N_NODES,), jnp.int32),
    ],
    compiler_params=pltpu.CompilerParams(needs_layout_passes=False),
)


def kernel(scores, particle_number, edge_index):
    scores_i32 = lax.bitcast_convert_type(scores.reshape(-1), jnp.int32)
    pn_i32 = particle_number.astype(jnp.int32)
    score_neigh, mask_i32 = _sc_call(scores_i32, pn_i32, edge_index.reshape(-1))
    return score_neigh, mask_i32.astype(jnp.bool_)
